# Initial kernel scaffold; baseline (speedup 1.0000x reference)
import functools

import jax
import jax.numpy as jnp
from jax import lax
from jax.experimental import pallas as pl
from jax.experimental.pallas import tpu as pltpu
from jax.experimental.pallas import tpu_sc as plsc

E = 16
NC, NS = 2, 16
NW = NC * NS


@functools.lru_cache(maxsize=None)
def _pool_sc(B, L, V):
    RW = B // NW
    C = 16
    NCHUNK = RW // C
    IDX = C * L
    NG = IDX // 128
    UN = 8

    mesh = plsc.VectorSubcoreMesh(core_axis_name="c", subcore_axis_name="s")

    @functools.partial(
        pl.kernel,
        out_type=jax.ShapeDtypeStruct((B, E), jnp.float32),
        mesh=mesh,
        scratch_types=[
            pltpu.VMEM((NG, 128), jnp.int32),
            pltpu.VMEM((IDX, E), jnp.float32),
            pltpu.VMEM((C, E), jnp.float32),
            pltpu.SemaphoreType.DMA,
        ],
    )
    def pool(x_hbm, table_hbm, out_hbm, idx_v, rows_v, pooled_v, sem):
        wid = lax.axis_index("c") * NS + lax.axis_index("s")

        def chunk_body(ci, carry):
            row0 = wid * RW + ci * C
            idx_off = row0 * (L // 128)
            pltpu.sync_copy(x_hbm.at[pl.ds(idx_off, NG)], idx_v)
            copies = [
                pltpu.async_copy(
                    table_hbm.at[idx_v.at[j]],
                    rows_v.at[pl.ds(j * 128, 128)],
                    sem,
                )
                for j in range(NG)
            ]
            for cp in copies:
                cp.wait()

            def row_body(r, c2):
                base = r * L

                def acc_body(i, accs):
                    o = base + i * UN
                    return tuple(accs[k] + rows_v[o + k] for k in range(UN))

                accs = lax.fori_loop(
                    0, L // UN, acc_body,
                    tuple(jnp.zeros((E,), jnp.float32) for _ in range(UN)),
                )
                s = accs[0]
                for k in range(1, UN):
                    s = s + accs[k]
                pooled_v[r] = s * (1.0 / L)
                return c2

            lax.fori_loop(0, C, row_body, 0)
            pltpu.sync_copy(pooled_v, out_hbm.at[pl.ds(row0, C)])
            return carry

        lax.fori_loop(0, NCHUNK, chunk_body, 0)

    return pool


def _mlp_body(p_ref, w1_ref, b1_ref, w2_ref, b2_ref, o_ref):
    p = p_ref[...]
    h = jnp.dot(p, w1_ref[...].T, preferred_element_type=jnp.float32)
    h = jnp.maximum(h + b1_ref[...], 0.0)
    o_ref[...] = jnp.dot(h, w2_ref[...].T, preferred_element_type=jnp.float32) + b2_ref[...]


def kernel(x, table, W1, b1, W2, b2):
    B, L = x.shape
    V, _ = table.shape
    x2 = x.reshape(B * L // 128, 128)
    pooled = _pool_sc(B, L, V)(x2, table)
    nc = W2.shape[0]
    out = pl.pallas_call(
        _mlp_body,
        out_shape=jax.ShapeDtypeStruct((B, nc), jnp.float32),
    )(pooled, W1, b1.reshape(1, -1), W2, b2.reshape(1, -1))
    return out


# speedup vs baseline: 8.1696x; 8.1696x over previous
import functools

import jax
import jax.numpy as jnp
from jax import lax
from jax.experimental import pallas as pl
from jax.experimental.pallas import tpu as pltpu
from jax.experimental.pallas import tpu_sc as plsc

E = 16
NC, NS = 2, 16
NW = NC * NS


@functools.lru_cache(maxsize=None)
def _pool_sc(B, L, V):
    RW = B // NW
    C = 16
    NCHUNK = RW // C
    IDX = C * L
    NG = IDX // 128
    UN = 8

    mesh = plsc.VectorSubcoreMesh(core_axis_name="c", subcore_axis_name="s")

    @functools.partial(
        pl.kernel,
        out_type=jax.ShapeDtypeStruct((B, E), jnp.float32),
        mesh=mesh,
        scratch_types=[
            pltpu.VMEM((IDX,), jnp.int32),
            pltpu.VMEM((IDX, E), jnp.float32),
            pltpu.VMEM((C, E), jnp.float32),
            pltpu.SemaphoreType.DMA,
        ],
        compiler_params=pltpu.CompilerParams(use_tc_tiling_on_sc=False),
    )
    def pool(x_hbm, table_hbm, out_hbm, idx_v, rows_v, pooled_v, sem):
        wid = lax.axis_index("c") * NS + lax.axis_index("s")

        def chunk_body(ci, carry):
            row0 = wid * RW + ci * C
            pltpu.sync_copy(x_hbm.at[pl.ds(row0 * L, IDX)], idx_v)
            copies = [
                pltpu.async_copy(
                    table_hbm.at[idx_v.at[pl.ds(j * 128, 128)]],
                    rows_v.at[pl.ds(j * 128, 128)],
                    sem,
                )
                for j in range(NG)
            ]
            for cp in copies:
                cp.wait()

            def row_body(r, c2):
                base = r * L

                def acc_body(i, accs):
                    o = base + i * UN
                    return tuple(accs[k] + rows_v[o + k] for k in range(UN))

                accs = lax.fori_loop(
                    0, L // UN, acc_body,
                    tuple(jnp.zeros((E,), jnp.float32) for _ in range(UN)),
                )
                s = accs[0]
                for k in range(1, UN):
                    s = s + accs[k]
                pooled_v[r] = s * (1.0 / L)
                return c2

            lax.fori_loop(0, C, row_body, 0)
            pltpu.sync_copy(pooled_v, out_hbm.at[pl.ds(row0, C)])
            return carry

        lax.fori_loop(0, NCHUNK, chunk_body, 0)

    return pool


def _mlp_body(p_ref, w1_ref, b1_ref, w2_ref, b2_ref, o_ref):
    p = p_ref[...]
    h = jnp.dot(p, w1_ref[...].T, preferred_element_type=jnp.float32)
    h = jnp.maximum(h + b1_ref[...], 0.0)
    o_ref[...] = jnp.dot(h, w2_ref[...].T, preferred_element_type=jnp.float32) + b2_ref[...]


def kernel(x, table, W1, b1, W2, b2):
    B, L = x.shape
    V, _ = table.shape
    x2 = x.reshape(B * L)
    pooled = _pool_sc(B, L, V)(x2, table)
    nc = W2.shape[0]
    out = pl.pallas_call(
        _mlp_body,
        out_shape=jax.ShapeDtypeStruct((B, nc), jnp.float32),
    )(pooled, W1, b1.reshape(1, -1), W2, b2.reshape(1, -1))
    return out


# speedup vs baseline: 8.1756x; 1.0007x over previous
import functools

import jax
import jax.numpy as jnp
from jax import lax
from jax.experimental import pallas as pl
from jax.experimental.pallas import tpu as pltpu
from jax.experimental.pallas import tpu_sc as plsc

E = 16
NC, NS = 2, 16
NW = NC * NS


@functools.lru_cache(maxsize=None)
def _pool_sc(B, L, V):
    RW = B // NW
    C = 16
    NCHUNK = RW // C
    IDX = C * L
    NG = IDX // 128
    UN = 8

    mesh = plsc.VectorSubcoreMesh(core_axis_name="c", subcore_axis_name="s")

    @functools.partial(
        pl.kernel,
        out_type=jax.ShapeDtypeStruct((B, E), jnp.float32),
        mesh=mesh,
        scratch_types=[
            pltpu.VMEM((IDX,), jnp.int32),
            pltpu.VMEM((IDX, E), jnp.float32),
            pltpu.VMEM((C, E), jnp.float32),
            pltpu.SemaphoreType.DMA,
        ],
        compiler_params=pltpu.CompilerParams(use_tc_tiling_on_sc=False),
    )
    def pool(x_hbm, table_hbm, out_hbm, idx_v, rows_v, pooled_v, sem):
        wid = lax.axis_index("c") * NS + lax.axis_index("s")

        def chunk_body(ci, carry):
            row0 = wid * RW + ci * C
            pltpu.sync_copy(x_hbm.at[pl.ds(row0 * L, IDX)], idx_v)
            copies = [
                pltpu.async_copy(
                    table_hbm.at[idx_v.at[pl.ds(j * 128, 128)]],
                    rows_v.at[pl.ds(j * 128, 128)],
                    sem,
                )
                for j in range(NG)
            ]
            for cp in copies:
                cp.wait()

            def row_body(r, c2):
                base = r * L

                def acc_body(i, accs):
                    o = base + i * UN
                    return tuple(accs[k] + rows_v[o + k] for k in range(UN))

                accs = lax.fori_loop(
                    0, L // UN, acc_body,
                    tuple(jnp.zeros((E,), jnp.float32) for _ in range(UN)),
                )
                s = accs[0]
                for k in range(1, UN):
                    s = s + accs[k]
                pooled_v[r] = s * (1.0 / L)
                return c2

            lax.fori_loop(0, C, row_body, 0)
            pltpu.sync_copy(pooled_v, out_hbm.at[pl.ds(row0, C)])
            return carry

        lax.fori_loop(0, NCHUNK, chunk_body, 0)

    return pool


def _mlp_body(p_ref, w1_ref, b1_ref, w2_ref, b2_ref, o_ref):
    p = p_ref[...]
    h = jnp.dot(p, w1_ref[...].T, preferred_element_type=jnp.float32)
    h = jnp.maximum(h + b1_ref[...], 0.0)
    o_ref[...] = jnp.dot(h, w2_ref[...].T, preferred_element_type=jnp.float32) + b2_ref[...]


def kernel(x, table, W1, b1, W2, b2):
    B, L = x.shape
    V, Ed = table.shape
    x2 = x.reshape(B * L)
    t2 = jax.lax.optimization_barrier(table.reshape(V // 8, 8 * Ed))
    t3 = t2.reshape(V, Ed)
    pooled = _pool_sc(B, L, V)(x2, t3)
    nc = W2.shape[0]
    out = pl.pallas_call(
        _mlp_body,
        out_shape=jax.ShapeDtypeStruct((B, nc), jnp.float32),
    )(pooled, W1, b1.reshape(1, -1), W2, b2.reshape(1, -1))
    return out
